# baseline (speedup 1.0000x reference)
import functools

import jax
import jax.numpy as jnp
from jax import lax
from jax.experimental import pallas as pl
from jax.experimental.pallas import tpu as pltpu
from jax.experimental.pallas import tpu_sc as plsc

N = 10000
E = 320000
D = 128
H = 128
HH = H // 2
NC = 2
NS = 16
NW = NC * NS
CH = 128
G = 80
E2 = NW * G * CH
NPAD = 10112
RPW = NPAD // NS

_MESH = plsc.VectorSubcoreMesh(
    core_axis_name="c", subcore_axis_name="s", num_cores=NC, num_subcores=NS)


@functools.partial(
    pl.kernel,
    out_type=jax.ShapeDtypeStruct((NC, NPAD, 16), jnp.float32),
    mesh=_MESH,
    compiler_params=pltpu.CompilerParams(use_tc_tiling_on_sc=False),
    scratch_types=[
        pltpu.VMEM((G, CH), jnp.int32),
        pltpu.VMEM((CH, 16), jnp.float32),
        pltpu.VMEM_SHARED((NPAD, 16), jnp.float32),
        pltpu.SemaphoreType.DMA,
    ],
)
def _deg_kernel(dst_hbm, zeros_hbm, ones_hbm, out_hbm, dst_v, ones_v, accum,
                sem):
    c = lax.axis_index("c")
    s = lax.axis_index("s")
    wid = s * NC + c
    pltpu.sync_copy(zeros_hbm.at[pl.ds(s * RPW, RPW)],
                    accum.at[pl.ds(s * RPW, RPW)])
    pltpu.sync_copy(dst_hbm.at[wid], dst_v)
    pltpu.sync_copy(ones_hbm, ones_v)
    plsc.subcore_barrier()

    def fire(j, carry):
        pltpu.async_copy(ones_v, accum.at[dst_v.at[j]], sem, add=True)
        return carry

    lax.fori_loop(0, G, fire, 0)

    def drain(j, carry):
        pltpu.make_async_copy(ones_v, accum.at[dst_v.at[0]], sem).wait()
        return carry

    lax.fori_loop(0, G, drain, 0)
    plsc.subcore_barrier()
    pltpu.sync_copy(accum.at[pl.ds(s * RPW, RPW)],
                    out_hbm.at[c, pl.ds(s * RPW, RPW)])


def _chunk_loop(hs_hbm, src_v, dst_v, accum, rows, semg, sems):
    pltpu.async_copy(hs_hbm.at[src_v.at[0]], rows[0], semg[0])

    def body(i, carry):
        j0 = 2 * i
        j1 = j0 + 1
        pltpu.async_copy(hs_hbm.at[src_v.at[j1]], rows[1], semg[1])
        pltpu.make_async_copy(hs_hbm.at[src_v.at[j0]], rows[0],
                              semg[0]).wait()
        pltpu.sync_copy(rows[0], accum.at[dst_v.at[j0]], add=True)

        @pl.when(j1 + 1 < G)
        def _():
            pltpu.async_copy(hs_hbm.at[src_v.at[j1 + 1]], rows[0], semg[0])

        pltpu.make_async_copy(hs_hbm.at[src_v.at[j1]], rows[1],
                              semg[1]).wait()
        pltpu.sync_copy(rows[1], accum.at[dst_v.at[j1]], add=True)
        return carry

    lax.fori_loop(0, G // 2, body, 0)


@functools.partial(
    pl.kernel,
    out_type=jax.ShapeDtypeStruct((NC, 2, NPAD, HH), jnp.float32),
    mesh=_MESH,
    compiler_params=pltpu.CompilerParams(use_tc_tiling_on_sc=False),
    scratch_types=[
        pltpu.VMEM((G, CH), jnp.int32),
        pltpu.VMEM((G, CH), jnp.int32),
        pltpu.VMEM((CH, HH), jnp.float32),
        pltpu.VMEM((CH, HH), jnp.float32),
        pltpu.VMEM((CH, HH), jnp.float32),
        pltpu.VMEM((CH, HH), jnp.float32),
        pltpu.VMEM_SHARED((NPAD, HH), jnp.float32),
        pltpu.SemaphoreType.DMA,
        pltpu.SemaphoreType.DMA,
        pltpu.SemaphoreType.DMA,
        pltpu.SemaphoreType.DMA,
        pltpu.SemaphoreType.DMA,
        pltpu.SemaphoreType.DMA,
        pltpu.SemaphoreType.DMA,
        pltpu.SemaphoreType.DMA,
    ],
)
def _seg_kernel(hsa_hbm, hsb_hbm, src_hbm, dst_hbm, zeros_hbm, out_hbm,
                src_v, dst_v, r0, r1, r2, r3, accum,
                sg0, sg1, sg2, sg3, ss0, ss1, ss2, ss3):
    c = lax.axis_index("c")
    s = lax.axis_index("s")
    wid = s * NC + c
    rows = [r0, r1, r2, r3]
    semg = [sg0, sg1, sg2, sg3]
    sems = [ss0, ss1, ss2, ss3]
    stripe = pl.ds(s * RPW, RPW)

    pltpu.sync_copy(zeros_hbm.at[stripe], accum.at[stripe])
    pltpu.sync_copy(src_hbm.at[wid], src_v)
    pltpu.sync_copy(dst_hbm.at[wid], dst_v)
    plsc.subcore_barrier()

    _chunk_loop(hsa_hbm, src_v, dst_v, accum, rows, semg, sems)
    plsc.subcore_barrier()
    pltpu.sync_copy(accum.at[stripe], out_hbm.at[c, 0, stripe])
    pltpu.sync_copy(zeros_hbm.at[stripe], accum.at[stripe])
    plsc.subcore_barrier()

    _chunk_loop(hsb_hbm, src_v, dst_v, accum, rows, semg, sems)
    plsc.subcore_barrier()
    pltpu.sync_copy(accum.at[stripe], out_hbm.at[c, 1, stripe])


def _pre_body(degp, x, w, dinv_o, hsa_o, hsb_o):
    dp = degp[...]
    dsum = dp[0, :N, 0:1] + dp[1, :N, 0:1] + 1.0
    dinv = lax.rsqrt(dsum)
    dinv_o[...] = dinv
    hs = dinv * jnp.dot(x[...], w[...], preferred_element_type=jnp.float32)
    hsa_o[...] = hs[:, :HH]
    hsb_o[...] = hs[:, HH:]


def _pre_call(degp, x, w):
    return pl.pallas_call(
        _pre_body,
        out_shape=(jax.ShapeDtypeStruct((N, 1), jnp.float32),
                   jax.ShapeDtypeStruct((N, HH), jnp.float32),
                   jax.ShapeDtypeStruct((N, HH), jnp.float32)),
    )(degp, x, w)


def _bn_relu(p, hsa, hsb, dinv, b, g, be):
    ta = p[0, 0, :N] + p[1, 0, :N] + hsa[...]
    tb = p[0, 1, :N] + p[1, 1, :N] + hsb[...]
    t = jnp.concatenate([ta, tb], axis=1)
    t = dinv[...] * t + b[...][None, :]
    mu = jnp.mean(t, axis=0, keepdims=True)
    var = jnp.mean((t - mu) ** 2, axis=0, keepdims=True)
    r = (t - mu) * lax.rsqrt(var + 1e-5) * g[...][None, :] + be[...][None, :]
    return jnp.maximum(r, 0.0)


def _mid_body(p, hsa, hsb, dinv, b, g, be, w, hsa_o, hsb_o):
    r = _bn_relu(p, hsa, hsb, dinv, b, g, be)
    hs = dinv[...] * jnp.dot(r, w[...], preferred_element_type=jnp.float32)
    hsa_o[...] = hs[:, :HH]
    hsb_o[...] = hs[:, HH:]


def _mid_call(p, hsa, hsb, dinv, b, g, be, w):
    return pl.pallas_call(
        _mid_body,
        out_shape=(jax.ShapeDtypeStruct((N, HH), jnp.float32),
                   jax.ShapeDtypeStruct((N, HH), jnp.float32)),
    )(p, hsa, hsb, dinv, b, g, be, w)


def _post_body(p, hsa, hsb, dinv, b, g, be, out):
    out[...] = _bn_relu(p, hsa, hsb, dinv, b, g, be)


def _post_call(p, hsa, hsb, dinv, b, g, be):
    return pl.pallas_call(
        _post_body,
        out_shape=jax.ShapeDtypeStruct((N, H), jnp.float32),
    )(p, hsa, hsb, dinv, b, g, be)


def kernel(x, edge_index, W1, b1, g1, be1, W2, b2, g2, be2, W3, b3, g3, be3):
    npad_e = E2 - E
    src = jnp.concatenate(
        [edge_index[0], jnp.zeros((npad_e,), jnp.int32)]).reshape(NW, G, CH)
    dst = jnp.concatenate(
        [edge_index[1], jnp.full((npad_e,), N, jnp.int32)]).reshape(NW, G, CH)
    zeros_nh = jnp.zeros((NPAD, HH), jnp.float32)
    zeros_n16 = jnp.zeros((NPAD, 16), jnp.float32)
    ones_c16 = jnp.ones((CH, 16), jnp.float32)

    degp = _deg_kernel(dst, zeros_n16, ones_c16)
    dinv, hsa, hsb = _pre_call(degp, x, W1)

    for (b, g, be, w) in ((b1, g1, be1, W2), (b2, g2, be2, W3)):
        p = _seg_kernel(hsa, hsb, src, dst, zeros_nh)
        hsa, hsb = _mid_call(p, hsa, hsb, dinv, b, g, be, w)

    p = _seg_kernel(hsa, hsb, src, dst, zeros_nh)
    return _post_call(p, hsa, hsb, dinv, b3, g3, be3)


# speedup vs baseline: 1.0176x; 1.0176x over previous
import functools

import jax
import jax.numpy as jnp
from jax import lax
from jax.experimental import pallas as pl
from jax.experimental.pallas import tpu as pltpu
from jax.experimental.pallas import tpu_sc as plsc

N = 10000
E = 320000
D = 128
H = 128
HH = H // 2
NC = 2
NS = 16
NW = NC * NS
CH = 128
G = 80
E2 = NW * G * CH
NPAD = 10112
RPW = NPAD // NS

_MESH = plsc.VectorSubcoreMesh(
    core_axis_name="c", subcore_axis_name="s", num_cores=NC, num_subcores=NS)


@functools.partial(
    pl.kernel,
    out_type=jax.ShapeDtypeStruct((NC, NPAD, 16), jnp.float32),
    mesh=_MESH,
    compiler_params=pltpu.CompilerParams(use_tc_tiling_on_sc=False),
    scratch_types=[
        pltpu.VMEM((G, CH), jnp.int32),
        pltpu.VMEM((CH, 16), jnp.float32),
        pltpu.VMEM_SHARED((NPAD, 16), jnp.float32),
        pltpu.SemaphoreType.DMA,
    ],
)
def _deg_kernel(dst_hbm, zeros_hbm, ones_hbm, out_hbm, dst_v, ones_v, accum,
                sem):
    c = lax.axis_index("c")
    s = lax.axis_index("s")
    wid = s * NC + c
    pltpu.sync_copy(zeros_hbm.at[pl.ds(s * RPW, RPW)],
                    accum.at[pl.ds(s * RPW, RPW)])
    pltpu.sync_copy(dst_hbm.at[wid], dst_v)
    pltpu.sync_copy(ones_hbm, ones_v)
    plsc.subcore_barrier()

    def fire(j, carry):
        pltpu.async_copy(ones_v, accum.at[dst_v.at[j]], sem, add=True)
        return carry

    lax.fori_loop(0, G, fire, 0)

    def drain(j, carry):
        pltpu.make_async_copy(ones_v, accum.at[dst_v.at[0]], sem).wait()
        return carry

    lax.fori_loop(0, G, drain, 0)
    plsc.subcore_barrier()
    pltpu.sync_copy(accum.at[pl.ds(s * RPW, RPW)],
                    out_hbm.at[c, pl.ds(s * RPW, RPW)])


def _chunk_loop(hs_hbm, src_v, dst_v, accum, rows, semg, sems):
    pltpu.async_copy(hs_hbm.at[src_v.at[0]], rows[0], semg[0])

    def body(i, carry):
        j0 = 2 * i
        j1 = j0 + 1
        pltpu.async_copy(hs_hbm.at[src_v.at[j1]], rows[1], semg[1])
        pltpu.make_async_copy(hs_hbm.at[src_v.at[j0]], rows[0],
                              semg[0]).wait()
        pltpu.sync_copy(rows[0], accum.at[dst_v.at[j0]], add=True)

        @pl.when(j1 + 1 < G)
        def _():
            pltpu.async_copy(hs_hbm.at[src_v.at[j1 + 1]], rows[0], semg[0])

        pltpu.make_async_copy(hs_hbm.at[src_v.at[j1]], rows[1],
                              semg[1]).wait()
        pltpu.sync_copy(rows[1], accum.at[dst_v.at[j1]], add=True)
        return carry

    lax.fori_loop(0, G // 2, body, 0)


@functools.partial(
    pl.kernel,
    out_type=jax.ShapeDtypeStruct((NC, 2, NPAD, HH), jnp.float32),
    mesh=_MESH,
    compiler_params=pltpu.CompilerParams(use_tc_tiling_on_sc=False),
    scratch_types=[
        pltpu.VMEM((G, CH), jnp.int32),
        pltpu.VMEM((G, CH), jnp.int32),
        pltpu.VMEM((CH, HH), jnp.float32),
        pltpu.VMEM((CH, HH), jnp.float32),
        pltpu.VMEM((CH, HH), jnp.float32),
        pltpu.VMEM((CH, HH), jnp.float32),
        pltpu.VMEM_SHARED((NPAD, HH), jnp.float32),
        pltpu.SemaphoreType.DMA,
        pltpu.SemaphoreType.DMA,
        pltpu.SemaphoreType.DMA,
        pltpu.SemaphoreType.DMA,
        pltpu.SemaphoreType.DMA,
        pltpu.SemaphoreType.DMA,
        pltpu.SemaphoreType.DMA,
        pltpu.SemaphoreType.DMA,
    ],
)
def _seg_kernel(hsa_hbm, hsb_hbm, src_hbm, dst_hbm, zeros_hbm, out_hbm,
                src_v, dst_v, r0, r1, r2, r3, accum,
                sg0, sg1, sg2, sg3, ss0, ss1, ss2, ss3):
    c = lax.axis_index("c")
    s = lax.axis_index("s")
    wid = s * NC + c
    rows = [r0, r1, r2, r3]
    semg = [sg0, sg1, sg2, sg3]
    sems = [ss0, ss1, ss2, ss3]
    stripe = pl.ds(s * RPW, RPW)

    pltpu.sync_copy(zeros_hbm.at[stripe], accum.at[stripe])
    pltpu.sync_copy(src_hbm.at[wid], src_v)
    pltpu.sync_copy(dst_hbm.at[wid], dst_v)
    plsc.subcore_barrier()

    _chunk_loop(hsa_hbm, src_v, dst_v, accum, rows, semg, sems)
    plsc.subcore_barrier()
    pltpu.sync_copy(accum.at[stripe], out_hbm.at[c, 0, stripe])
    pltpu.sync_copy(zeros_hbm.at[stripe], accum.at[stripe])
    plsc.subcore_barrier()

    _chunk_loop(hsb_hbm, src_v, dst_v, accum, rows, semg, sems)
    plsc.subcore_barrier()
    pltpu.sync_copy(accum.at[stripe], out_hbm.at[c, 1, stripe])


def _pre_body(degp, x, w, dinv_o, hsa_o, hsb_o):
    dp = degp[...]
    dsum = dp[0, :N, 0:1] + dp[1, :N, 0:1] + 1.0
    dinv = lax.rsqrt(dsum)
    dinv_o[...] = dinv
    hs = dinv * jnp.dot(x[...], w[...], preferred_element_type=jnp.float32)
    hsa_o[...] = hs[:, :HH]
    hsb_o[...] = hs[:, HH:]


def _pre_call(degp, x, w):
    return pl.pallas_call(
        _pre_body,
        out_shape=(jax.ShapeDtypeStruct((N, 1), jnp.float32),
                   jax.ShapeDtypeStruct((N, HH), jnp.float32),
                   jax.ShapeDtypeStruct((N, HH), jnp.float32)),
    )(degp, x, w)


def _bn_relu(p, hsa, hsb, dinv, b, g, be):
    ta = p[0, 0, :N] + p[1, 0, :N] + hsa[...]
    tb = p[0, 1, :N] + p[1, 1, :N] + hsb[...]
    t = jnp.concatenate([ta, tb], axis=1)
    t = dinv[...] * t + b[...][None, :]
    mu = jnp.mean(t, axis=0, keepdims=True)
    var = jnp.mean((t - mu) ** 2, axis=0, keepdims=True)
    r = (t - mu) * lax.rsqrt(var + 1e-5) * g[...][None, :] + be[...][None, :]
    return jnp.maximum(r, 0.0)


def _mid_body(p, hsa, hsb, dinv, b, g, be, w, hsa_o, hsb_o):
    r = _bn_relu(p, hsa, hsb, dinv, b, g, be)
    hs = dinv[...] * jnp.dot(r, w[...], preferred_element_type=jnp.float32)
    hsa_o[...] = hs[:, :HH]
    hsb_o[...] = hs[:, HH:]


def _mid_call(p, hsa, hsb, dinv, b, g, be, w):
    return pl.pallas_call(
        _mid_body,
        out_shape=(jax.ShapeDtypeStruct((N, HH), jnp.float32),
                   jax.ShapeDtypeStruct((N, HH), jnp.float32)),
    )(p, hsa, hsb, dinv, b, g, be, w)


def _post_body(p, hsa, hsb, dinv, b, g, be, out):
    out[...] = _bn_relu(p, hsa, hsb, dinv, b, g, be)


def _post_call(p, hsa, hsb, dinv, b, g, be):
    return pl.pallas_call(
        _post_body,
        out_shape=jax.ShapeDtypeStruct((N, H), jnp.float32),
    )(p, hsa, hsb, dinv, b, g, be)


def kernel(x, edge_index, W1, b1, g1, be1, W2, b2, g2, be2, W3, b3, g3, be3):
    npad_e = E2 - E
    src = jnp.concatenate(
        [edge_index[0], jnp.zeros((npad_e,), jnp.int32)]).reshape(NW, G, CH)
    pad_dst = N + jnp.arange(npad_e, dtype=jnp.int32) % (NPAD - N)
    dst = jnp.concatenate([edge_index[1], pad_dst]).reshape(NW, G, CH)
    zeros_nh = jnp.zeros((NPAD, HH), jnp.float32)
    zeros_n16 = jnp.zeros((NPAD, 16), jnp.float32)
    ones_c16 = jnp.ones((CH, 16), jnp.float32)

    degp = _deg_kernel(dst, zeros_n16, ones_c16)
    dinv, hsa, hsb = _pre_call(degp, x, W1)

    for (b, g, be, w) in ((b1, g1, be1, W2), (b2, g2, be2, W3)):
        p = _seg_kernel(hsa, hsb, src, dst, zeros_nh)
        hsa, hsb = _mid_call(p, hsa, hsb, dinv, b, g, be, w)

    p = _seg_kernel(hsa, hsb, src, dst, zeros_nh)
    return _post_call(p, hsa, hsb, dinv, b3, g3, be3)


# speedup vs baseline: 1.0565x; 1.0382x over previous
import functools

import jax
import jax.numpy as jnp
from jax import lax
from jax.experimental import pallas as pl
from jax.experimental.pallas import tpu as pltpu
from jax.experimental.pallas import tpu_sc as plsc

N = 10000
E = 320000
D = 128
H = 128
HH = H // 2
NC = 2
NS = 16
NW = NC * NS
CH = 128
G = 80
E2 = NW * G * CH
NPAD = 10112
RPW = NPAD // NS

_MESH = plsc.VectorSubcoreMesh(
    core_axis_name="c", subcore_axis_name="s", num_cores=NC, num_subcores=NS)


@functools.partial(
    pl.kernel,
    out_type=jax.ShapeDtypeStruct((NC, NPAD, 16), jnp.float32),
    mesh=_MESH,
    compiler_params=pltpu.CompilerParams(use_tc_tiling_on_sc=False),
    scratch_types=[
        pltpu.VMEM((G, CH), jnp.int32),
        pltpu.VMEM((CH, 16), jnp.float32),
        pltpu.VMEM_SHARED((NPAD, 16), jnp.float32),
        pltpu.SemaphoreType.DMA,
    ],
)
def _deg_kernel(dst_hbm, zeros_hbm, ones_hbm, out_hbm, dst_v, ones_v, accum,
                sem):
    c = lax.axis_index("c")
    s = lax.axis_index("s")
    wid = s * NC + c
    pltpu.sync_copy(zeros_hbm.at[pl.ds(s * RPW, RPW)],
                    accum.at[pl.ds(s * RPW, RPW)])
    pltpu.sync_copy(dst_hbm.at[wid], dst_v)
    pltpu.sync_copy(ones_hbm, ones_v)
    plsc.subcore_barrier()

    def fire(j, carry):
        pltpu.async_copy(ones_v, accum.at[dst_v.at[j]], sem, add=True)
        return carry

    lax.fori_loop(0, G, fire, 0)

    def drain(j, carry):
        pltpu.make_async_copy(ones_v, accum.at[dst_v.at[0]], sem).wait()
        return carry

    lax.fori_loop(0, G, drain, 0)
    plsc.subcore_barrier()
    pltpu.sync_copy(accum.at[pl.ds(s * RPW, RPW)],
                    out_hbm.at[c, pl.ds(s * RPW, RPW)])


def _chunk_loop(hs_hbm, src_v, dst_v, accum, rows, semg, sems):
    pltpu.async_copy(hs_hbm.at[src_v.at[0]], rows[0], semg[0])

    def body(i, carry):
        j0 = 2 * i
        j1 = j0 + 1
        pltpu.async_copy(hs_hbm.at[src_v.at[j1]], rows[1], semg[1])
        pltpu.make_async_copy(hs_hbm.at[src_v.at[j0]], rows[0],
                              semg[0]).wait()
        pltpu.sync_copy(rows[0], accum.at[dst_v.at[j0]], add=True)

        @pl.when(j1 + 1 < G)
        def _():
            pltpu.async_copy(hs_hbm.at[src_v.at[j1 + 1]], rows[0], semg[0])

        pltpu.make_async_copy(hs_hbm.at[src_v.at[j1]], rows[1],
                              semg[1]).wait()
        pltpu.sync_copy(rows[1], accum.at[dst_v.at[j1]], add=True)
        return carry

    lax.fori_loop(0, G // 2, body, 0)


@functools.partial(
    pl.kernel,
    out_type=jax.ShapeDtypeStruct((NC, NPAD, HH), jnp.float32),
    mesh=_MESH,
    compiler_params=pltpu.CompilerParams(use_tc_tiling_on_sc=False),
    scratch_types=[
        pltpu.VMEM((G, CH), jnp.int32),
        pltpu.VMEM((G, CH), jnp.int32),
        pltpu.VMEM((CH, HH), jnp.float32),
        pltpu.VMEM((CH, HH), jnp.float32),
        pltpu.VMEM((CH, HH), jnp.float32),
        pltpu.VMEM((CH, HH), jnp.float32),
        pltpu.VMEM_SHARED((NPAD, HH), jnp.float32),
        pltpu.SemaphoreType.DMA,
        pltpu.SemaphoreType.DMA,
        pltpu.SemaphoreType.DMA,
        pltpu.SemaphoreType.DMA,
        pltpu.SemaphoreType.DMA,
        pltpu.SemaphoreType.DMA,
        pltpu.SemaphoreType.DMA,
        pltpu.SemaphoreType.DMA,
    ],
)
def _seg_kernel(hs_hbm, src_hbm, dst_hbm, zeros_hbm, out_hbm,
                src_v, dst_v, r0, r1, r2, r3, accum,
                sg0, sg1, sg2, sg3, ss0, ss1, ss2, ss3):
    c = lax.axis_index("c")
    s = lax.axis_index("s")
    wid = s * NC + c
    rows = [r0, r1, r2, r3]
    semg = [sg0, sg1, sg2, sg3]
    sems = [ss0, ss1, ss2, ss3]
    stripe = pl.ds(s * RPW, RPW)

    pltpu.sync_copy(zeros_hbm.at[stripe], accum.at[stripe])
    pltpu.sync_copy(src_hbm.at[wid], src_v)
    pltpu.sync_copy(dst_hbm.at[wid], dst_v)
    plsc.subcore_barrier()

    _chunk_loop(hs_hbm, src_v, dst_v, accum, rows, semg, sems)
    plsc.subcore_barrier()
    pltpu.sync_copy(accum.at[stripe], out_hbm.at[c, stripe])


def _pre_body(degp, x, w, dinv_o, hsa_o, hsb_o):
    dp = degp[...]
    dsum = dp[0, :N, 0:1] + dp[1, :N, 0:1] + 1.0
    dinv = lax.rsqrt(dsum)
    dinv_o[...] = dinv
    hs = dinv * jnp.dot(x[...], w[...], preferred_element_type=jnp.float32)
    hsa_o[...] = hs[:, :HH]
    hsb_o[...] = hs[:, HH:]


def _pre_call(degp, x, w):
    return pl.pallas_call(
        _pre_body,
        out_shape=(jax.ShapeDtypeStruct((N, 1), jnp.float32),
                   jax.ShapeDtypeStruct((N, HH), jnp.float32),
                   jax.ShapeDtypeStruct((N, HH), jnp.float32)),
    )(degp, x, w)


def _bn_relu(pa, pb, hsa, hsb, dinv, b, g, be):
    ta = pa[0, :N] + pa[1, :N] + hsa[...]
    tb = pb[0, :N] + pb[1, :N] + hsb[...]
    t = jnp.concatenate([ta, tb], axis=1)
    t = dinv[...] * t + b[...][None, :]
    mu = jnp.mean(t, axis=0, keepdims=True)
    var = jnp.mean((t - mu) ** 2, axis=0, keepdims=True)
    r = (t - mu) * lax.rsqrt(var + 1e-5) * g[...][None, :] + be[...][None, :]
    return jnp.maximum(r, 0.0)


def _mid_body(pa, pb, hsa, hsb, dinv, b, g, be, w, hsa_o, hsb_o):
    r = _bn_relu(pa, pb, hsa, hsb, dinv, b, g, be)
    hs = dinv[...] * jnp.dot(r, w[...], preferred_element_type=jnp.float32)
    hsa_o[...] = hs[:, :HH]
    hsb_o[...] = hs[:, HH:]


def _mid_call(pa, pb, hsa, hsb, dinv, b, g, be, w):
    return pl.pallas_call(
        _mid_body,
        out_shape=(jax.ShapeDtypeStruct((N, HH), jnp.float32),
                   jax.ShapeDtypeStruct((N, HH), jnp.float32)),
    )(pa, pb, hsa, hsb, dinv, b, g, be, w)


def _post_body(pa, pb, hsa, hsb, dinv, b, g, be, out):
    out[...] = _bn_relu(pa, pb, hsa, hsb, dinv, b, g, be)


def _post_call(pa, pb, hsa, hsb, dinv, b, g, be):
    return pl.pallas_call(
        _post_body,
        out_shape=jax.ShapeDtypeStruct((N, H), jnp.float32),
    )(pa, pb, hsa, hsb, dinv, b, g, be)


def kernel(x, edge_index, W1, b1, g1, be1, W2, b2, g2, be2, W3, b3, g3, be3):
    npad_e = E2 - E
    src = jnp.concatenate(
        [edge_index[0], jnp.zeros((npad_e,), jnp.int32)]).reshape(NW, G, CH)
    pad_dst = N + jnp.arange(npad_e, dtype=jnp.int32) % (NPAD - N)
    dst = jnp.concatenate([edge_index[1], pad_dst]).reshape(NW, G, CH)
    zeros_nh = jnp.zeros((NPAD, HH), jnp.float32)
    zeros_n16 = jnp.zeros((NPAD, 16), jnp.float32)
    ones_c16 = jnp.ones((CH, 16), jnp.float32)

    degp = _deg_kernel(dst, zeros_n16, ones_c16)
    dinv, hsa, hsb = _pre_call(degp, x, W1)

    for (b, g, be, w) in ((b1, g1, be1, W2), (b2, g2, be2, W3)):
        pa = _seg_kernel(hsa, src, dst, zeros_nh)
        pb = _seg_kernel(hsb, src, dst, zeros_nh)
        hsa, hsb = _mid_call(pa, pb, hsa, hsb, dinv, b, g, be, w)

    pa = _seg_kernel(hsa, src, dst, zeros_nh)
    pb = _seg_kernel(hsb, src, dst, zeros_nh)
    return _post_call(pa, pb, hsa, hsb, dinv, b3, g3, be3)


# speedup vs baseline: 2.7370x; 2.5907x over previous
import functools

import jax
import jax.numpy as jnp
from jax import lax
from jax.experimental import pallas as pl
from jax.experimental.pallas import tpu as pltpu
from jax.experimental.pallas import tpu_sc as plsc

N = 10000
E = 320000
D = 128
H = 128
HH = H // 2
NC = 2
NS = 16
NW = NC * NS
CH = 125
G = 80
E2 = NW * G * CH
NPAD = 10112
RPW = NPAD // NS

_MESH = plsc.VectorSubcoreMesh(
    core_axis_name="c", subcore_axis_name="s", num_cores=NC, num_subcores=NS)


@functools.partial(
    pl.kernel,
    out_type=jax.ShapeDtypeStruct((NC, NPAD, 16), jnp.float32),
    mesh=_MESH,
    compiler_params=pltpu.CompilerParams(use_tc_tiling_on_sc=False),
    scratch_types=[
        pltpu.VMEM((G, CH), jnp.int32),
        pltpu.VMEM((CH, 16), jnp.float32),
        pltpu.VMEM_SHARED((NPAD, 16), jnp.float32),
        pltpu.SemaphoreType.DMA,
    ],
)
def _deg_kernel(dst_hbm, zeros_hbm, ones_hbm, out_hbm, dst_v, ones_v, accum,
                sem):
    c = lax.axis_index("c")
    s = lax.axis_index("s")
    wid = s * NC + c
    pltpu.sync_copy(zeros_hbm.at[pl.ds(s * RPW, RPW)],
                    accum.at[pl.ds(s * RPW, RPW)])
    pltpu.sync_copy(dst_hbm.at[wid], dst_v)
    pltpu.sync_copy(ones_hbm, ones_v)
    plsc.subcore_barrier()

    def fire(j, carry):
        pltpu.async_copy(ones_v, accum.at[dst_v.at[j]], sem, add=True)
        return carry

    lax.fori_loop(0, G, fire, 0)

    def drain(j, carry):
        pltpu.make_async_copy(ones_v, accum.at[dst_v.at[0]], sem).wait()
        return carry

    lax.fori_loop(0, G, drain, 0)
    plsc.subcore_barrier()
    pltpu.sync_copy(accum.at[pl.ds(s * RPW, RPW)],
                    out_hbm.at[c, pl.ds(s * RPW, RPW)])


def _chunk_loop(hs_hbm, src_v, dst_v, accum, rows, semg, sems):
    pltpu.async_copy(hs_hbm.at[src_v.at[0]], rows[0], semg[0])

    def body(i, carry):
        j0 = 2 * i
        j1 = j0 + 1
        pltpu.async_copy(hs_hbm.at[src_v.at[j1]], rows[1], semg[1])
        pltpu.make_async_copy(hs_hbm.at[src_v.at[j0]], rows[0],
                              semg[0]).wait()
        pltpu.sync_copy(rows[0], accum.at[dst_v.at[j0]], add=True)

        @pl.when(j1 + 1 < G)
        def _():
            pltpu.async_copy(hs_hbm.at[src_v.at[j1 + 1]], rows[0], semg[0])

        pltpu.make_async_copy(hs_hbm.at[src_v.at[j1]], rows[1],
                              semg[1]).wait()
        pltpu.sync_copy(rows[1], accum.at[dst_v.at[j1]], add=True)
        return carry

    lax.fori_loop(0, G // 2, body, 0)


@functools.partial(
    pl.kernel,
    out_type=jax.ShapeDtypeStruct((NC, NPAD, HH), jnp.float32),
    mesh=_MESH,
    compiler_params=pltpu.CompilerParams(use_tc_tiling_on_sc=False),
    scratch_types=[
        pltpu.VMEM((G, CH), jnp.int32),
        pltpu.VMEM((G, CH), jnp.int32),
        pltpu.VMEM((CH, HH), jnp.float32),
        pltpu.VMEM((CH, HH), jnp.float32),
        pltpu.VMEM((CH, HH), jnp.float32),
        pltpu.VMEM((CH, HH), jnp.float32),
        pltpu.VMEM_SHARED((NPAD, HH), jnp.float32),
        pltpu.SemaphoreType.DMA,
        pltpu.SemaphoreType.DMA,
        pltpu.SemaphoreType.DMA,
        pltpu.SemaphoreType.DMA,
        pltpu.SemaphoreType.DMA,
        pltpu.SemaphoreType.DMA,
        pltpu.SemaphoreType.DMA,
        pltpu.SemaphoreType.DMA,
    ],
)
def _seg_kernel(hs_hbm, src_hbm, dst_hbm, zeros_hbm, out_hbm,
                src_v, dst_v, r0, r1, r2, r3, accum,
                sg0, sg1, sg2, sg3, ss0, ss1, ss2, ss3):
    c = lax.axis_index("c")
    s = lax.axis_index("s")
    wid = s * NC + c
    rows = [r0, r1, r2, r3]
    semg = [sg0, sg1, sg2, sg3]
    sems = [ss0, ss1, ss2, ss3]
    stripe = pl.ds(s * RPW, RPW)

    pltpu.sync_copy(zeros_hbm.at[stripe], accum.at[stripe])
    pltpu.sync_copy(src_hbm.at[wid], src_v)
    pltpu.sync_copy(dst_hbm.at[wid], dst_v)
    plsc.subcore_barrier()

    _chunk_loop(hs_hbm, src_v, dst_v, accum, rows, semg, sems)
    plsc.subcore_barrier()
    pltpu.sync_copy(accum.at[stripe], out_hbm.at[c, stripe])


def _pre_body(degp, x, w, dinv_o, hsa_o, hsb_o):
    dp = degp[...]
    dsum = dp[0, :N, 0:1] + dp[1, :N, 0:1] + 1.0
    dinv = lax.rsqrt(dsum)
    dinv_o[...] = dinv
    hs = dinv * jnp.dot(x[...], w[...], preferred_element_type=jnp.float32)
    hsa_o[...] = hs[:, :HH]
    hsb_o[...] = hs[:, HH:]


def _pre_call(degp, x, w):
    return pl.pallas_call(
        _pre_body,
        out_shape=(jax.ShapeDtypeStruct((N, 1), jnp.float32),
                   jax.ShapeDtypeStruct((N, HH), jnp.float32),
                   jax.ShapeDtypeStruct((N, HH), jnp.float32)),
    )(degp, x, w)


def _bn_relu(pa, pb, hsa, hsb, dinv, b, g, be):
    ta = pa[0, :N] + pa[1, :N] + hsa[...]
    tb = pb[0, :N] + pb[1, :N] + hsb[...]
    t = jnp.concatenate([ta, tb], axis=1)
    t = dinv[...] * t + b[...][None, :]
    mu = jnp.mean(t, axis=0, keepdims=True)
    var = jnp.mean((t - mu) ** 2, axis=0, keepdims=True)
    r = (t - mu) * lax.rsqrt(var + 1e-5) * g[...][None, :] + be[...][None, :]
    return jnp.maximum(r, 0.0)


def _mid_body(pa, pb, hsa, hsb, dinv, b, g, be, w, hsa_o, hsb_o):
    r = _bn_relu(pa, pb, hsa, hsb, dinv, b, g, be)
    hs = dinv[...] * jnp.dot(r, w[...], preferred_element_type=jnp.float32)
    hsa_o[...] = hs[:, :HH]
    hsb_o[...] = hs[:, HH:]


def _mid_call(pa, pb, hsa, hsb, dinv, b, g, be, w):
    return pl.pallas_call(
        _mid_body,
        out_shape=(jax.ShapeDtypeStruct((N, HH), jnp.float32),
                   jax.ShapeDtypeStruct((N, HH), jnp.float32)),
    )(pa, pb, hsa, hsb, dinv, b, g, be, w)


def _post_body(pa, pb, hsa, hsb, dinv, b, g, be, out):
    out[...] = _bn_relu(pa, pb, hsa, hsb, dinv, b, g, be)


def _post_call(pa, pb, hsa, hsb, dinv, b, g, be):
    return pl.pallas_call(
        _post_body,
        out_shape=jax.ShapeDtypeStruct((N, H), jnp.float32),
    )(pa, pb, hsa, hsb, dinv, b, g, be)


def kernel(x, edge_index, W1, b1, g1, be1, W2, b2, g2, be2, W3, b3, g3, be3):
    src = edge_index[0].reshape(NW, G, CH)
    dst = edge_index[1].reshape(NW, G, CH)
    zeros_nh = jnp.zeros((NPAD, HH), jnp.float32)
    zeros_n16 = jnp.zeros((NPAD, 16), jnp.float32)
    ones_c16 = jnp.ones((CH, 16), jnp.float32)

    degp = _deg_kernel(dst, zeros_n16, ones_c16)
    dinv, hsa, hsb = _pre_call(degp, x, W1)

    for (b, g, be, w) in ((b1, g1, be1, W2), (b2, g2, be2, W3)):
        pa = _seg_kernel(hsa, src, dst, zeros_nh)
        pb = _seg_kernel(hsb, src, dst, zeros_nh)
        hsa, hsb = _mid_call(pa, pb, hsa, hsb, dinv, b, g, be, w)

    pa = _seg_kernel(hsa, src, dst, zeros_nh)
    pb = _seg_kernel(hsb, src, dst, zeros_nh)
    return _post_call(pa, pb, hsa, hsb, dinv, b3, g3, be3)


# speedup vs baseline: 2.8501x; 1.0413x over previous
import functools

import jax
import jax.numpy as jnp
from jax import lax
from jax.experimental import pallas as pl
from jax.experimental.pallas import tpu as pltpu
from jax.experimental.pallas import tpu_sc as plsc

N = 10000
E = 320000
D = 128
H = 128
HH = H // 2
NC = 2
NS = 16
NW = NC * NS
CH = 125
G = 80
E2 = NW * G * CH
NPAD = 10112
RPW = NPAD // NS

_MESH = plsc.VectorSubcoreMesh(
    core_axis_name="c", subcore_axis_name="s", num_cores=NC, num_subcores=NS)


@functools.partial(
    pl.kernel,
    out_type=jax.ShapeDtypeStruct((NC, NPAD, 16), jnp.float32),
    mesh=_MESH,
    compiler_params=pltpu.CompilerParams(use_tc_tiling_on_sc=False),
    scratch_types=[
        pltpu.VMEM((G, CH), jnp.int32),
        pltpu.VMEM((CH, 16), jnp.float32),
        pltpu.VMEM_SHARED((NPAD, 16), jnp.float32),
        pltpu.SemaphoreType.DMA,
    ],
)
def _deg_kernel(dst_hbm, zeros_hbm, ones_hbm, out_hbm, dst_v, ones_v, accum,
                sem):
    c = lax.axis_index("c")
    s = lax.axis_index("s")
    wid = s * NC + c
    pltpu.sync_copy(zeros_hbm.at[pl.ds(s * RPW, RPW)],
                    accum.at[pl.ds(s * RPW, RPW)])
    pltpu.sync_copy(dst_hbm.at[wid], dst_v)
    pltpu.sync_copy(ones_hbm, ones_v)
    plsc.subcore_barrier()

    def fire(j, carry):
        pltpu.async_copy(ones_v, accum.at[dst_v.at[j]], sem, add=True)
        return carry

    lax.fori_loop(0, G, fire, 0)

    def drain(j, carry):
        pltpu.make_async_copy(ones_v, accum.at[dst_v.at[0]], sem).wait()
        return carry

    lax.fori_loop(0, G, drain, 0)
    plsc.subcore_barrier()
    pltpu.sync_copy(accum.at[pl.ds(s * RPW, RPW)],
                    out_hbm.at[c, pl.ds(s * RPW, RPW)])


def _chunk_loop(hs_hbm, src_v, dst_v, accum, rows, semg, sems):
    pltpu.async_copy(hs_hbm.at[src_v.at[0]], rows[0], semg[0])
    pltpu.async_copy(hs_hbm.at[src_v.at[1]], rows[1], semg[1])

    def body(i, carry):
        for k in range(4):
            j = 4 * i + k
            k2 = (k + 2) % 4
            pltpu.make_async_copy(hs_hbm.at[src_v.at[j]], rows[k],
                                  semg[k]).wait()
            pltpu.async_copy(rows[k], accum.at[dst_v.at[j]], sems[k],
                             add=True)
            if k < 2:
                @pl.when(i > 0)
                def _():
                    pltpu.make_async_copy(rows[k2], accum.at[dst_v.at[0]],
                                          sems[k2]).wait()
            else:
                pltpu.make_async_copy(rows[k2], accum.at[dst_v.at[0]],
                                      sems[k2]).wait()

            @pl.when(j + 2 < G)
            def _():
                pltpu.async_copy(hs_hbm.at[src_v.at[j + 2]], rows[k2],
                                 semg[k2])
        return carry

    lax.fori_loop(0, G // 4, body, 0)
    for j in (G - 2, G - 1):
        k = j % 4
        pltpu.make_async_copy(rows[k], accum.at[dst_v.at[0]], sems[k]).wait()


@functools.partial(
    pl.kernel,
    out_type=jax.ShapeDtypeStruct((NC, NPAD, HH), jnp.float32),
    mesh=_MESH,
    compiler_params=pltpu.CompilerParams(use_tc_tiling_on_sc=False),
    scratch_types=[
        pltpu.VMEM((G, CH), jnp.int32),
        pltpu.VMEM((G, CH), jnp.int32),
        pltpu.VMEM((CH, HH), jnp.float32),
        pltpu.VMEM((CH, HH), jnp.float32),
        pltpu.VMEM((CH, HH), jnp.float32),
        pltpu.VMEM((CH, HH), jnp.float32),
        pltpu.VMEM_SHARED((NPAD, HH), jnp.float32),
        pltpu.SemaphoreType.DMA,
        pltpu.SemaphoreType.DMA,
        pltpu.SemaphoreType.DMA,
        pltpu.SemaphoreType.DMA,
        pltpu.SemaphoreType.DMA,
        pltpu.SemaphoreType.DMA,
        pltpu.SemaphoreType.DMA,
        pltpu.SemaphoreType.DMA,
    ],
)
def _seg_kernel(hs_hbm, src_hbm, dst_hbm, zeros_hbm, out_hbm,
                src_v, dst_v, r0, r1, r2, r3, accum,
                sg0, sg1, sg2, sg3, ss0, ss1, ss2, ss3):
    c = lax.axis_index("c")
    s = lax.axis_index("s")
    wid = s * NC + c
    rows = [r0, r1, r2, r3]
    semg = [sg0, sg1, sg2, sg3]
    sems = [ss0, ss1, ss2, ss3]
    stripe = pl.ds(s * RPW, RPW)

    pltpu.sync_copy(zeros_hbm.at[stripe], accum.at[stripe])
    pltpu.sync_copy(src_hbm.at[wid], src_v)
    pltpu.sync_copy(dst_hbm.at[wid], dst_v)
    plsc.subcore_barrier()

    _chunk_loop(hs_hbm, src_v, dst_v, accum, rows, semg, sems)
    plsc.subcore_barrier()
    pltpu.sync_copy(accum.at[stripe], out_hbm.at[c, stripe])


def _pre_body(degp, x, w, dinv_o, hsa_o, hsb_o):
    dp = degp[...]
    dsum = dp[0, :N, 0:1] + dp[1, :N, 0:1] + 1.0
    dinv = lax.rsqrt(dsum)
    dinv_o[...] = dinv
    hs = dinv * jnp.dot(x[...], w[...], preferred_element_type=jnp.float32)
    hsa_o[...] = hs[:, :HH]
    hsb_o[...] = hs[:, HH:]


def _pre_call(degp, x, w):
    return pl.pallas_call(
        _pre_body,
        out_shape=(jax.ShapeDtypeStruct((N, 1), jnp.float32),
                   jax.ShapeDtypeStruct((N, HH), jnp.float32),
                   jax.ShapeDtypeStruct((N, HH), jnp.float32)),
    )(degp, x, w)


def _bn_relu(pa, pb, hsa, hsb, dinv, b, g, be):
    ta = pa[0, :N] + pa[1, :N] + hsa[...]
    tb = pb[0, :N] + pb[1, :N] + hsb[...]
    t = jnp.concatenate([ta, tb], axis=1)
    t = dinv[...] * t + b[...][None, :]
    mu = jnp.mean(t, axis=0, keepdims=True)
    var = jnp.mean((t - mu) ** 2, axis=0, keepdims=True)
    r = (t - mu) * lax.rsqrt(var + 1e-5) * g[...][None, :] + be[...][None, :]
    return jnp.maximum(r, 0.0)


def _mid_body(pa, pb, hsa, hsb, dinv, b, g, be, w, hsa_o, hsb_o):
    r = _bn_relu(pa, pb, hsa, hsb, dinv, b, g, be)
    hs = dinv[...] * jnp.dot(r, w[...], preferred_element_type=jnp.float32)
    hsa_o[...] = hs[:, :HH]
    hsb_o[...] = hs[:, HH:]


def _mid_call(pa, pb, hsa, hsb, dinv, b, g, be, w):
    return pl.pallas_call(
        _mid_body,
        out_shape=(jax.ShapeDtypeStruct((N, HH), jnp.float32),
                   jax.ShapeDtypeStruct((N, HH), jnp.float32)),
    )(pa, pb, hsa, hsb, dinv, b, g, be, w)


def _post_body(pa, pb, hsa, hsb, dinv, b, g, be, out):
    out[...] = _bn_relu(pa, pb, hsa, hsb, dinv, b, g, be)


def _post_call(pa, pb, hsa, hsb, dinv, b, g, be):
    return pl.pallas_call(
        _post_body,
        out_shape=jax.ShapeDtypeStruct((N, H), jnp.float32),
    )(pa, pb, hsa, hsb, dinv, b, g, be)


def kernel(x, edge_index, W1, b1, g1, be1, W2, b2, g2, be2, W3, b3, g3, be3):
    src = edge_index[0].reshape(NW, G, CH)
    dst = edge_index[1].reshape(NW, G, CH)
    zeros_nh = jnp.zeros((NPAD, HH), jnp.float32)
    zeros_n16 = jnp.zeros((NPAD, 16), jnp.float32)
    ones_c16 = jnp.ones((CH, 16), jnp.float32)

    degp = _deg_kernel(dst, zeros_n16, ones_c16)
    dinv, hsa, hsb = _pre_call(degp, x, W1)

    for (b, g, be, w) in ((b1, g1, be1, W2), (b2, g2, be2, W3)):
        pa = _seg_kernel(hsa, src, dst, zeros_nh)
        pb = _seg_kernel(hsb, src, dst, zeros_nh)
        hsa, hsb = _mid_call(pa, pb, hsa, hsb, dinv, b, g, be, w)

    pa = _seg_kernel(hsa, src, dst, zeros_nh)
    pb = _seg_kernel(hsb, src, dst, zeros_nh)
    return _post_call(pa, pb, hsa, hsb, dinv, b3, g3, be3)
